# baseline (speedup 1.0000x reference)
import jax
import jax.numpy as jnp
from jax import lax
from jax.experimental import pallas as pl
from jax.experimental.pallas import tpu as pltpu
from jax.experimental.pallas import tpu_sc as plsc

B, L, D = 4096, 200, 64
V = 1_000_000
NC, NS = 2, 16
NW = NC * NS
VBLK = 7812
KSUP = 2
SBW = 128 * KSUP
NSUP = VBLK // KSUP
SUP_BASE = NSUP // NW
SUP_EXTRA = NSUP % NW
L_TILES = L // 8
BT = B // 128


def _wid():
    return lax.axis_index("s") * NC + lax.axis_index("c")


def _kernel_a(wt, wt_tail, wrm, sbuf, tbuf,
              ssem0, ssem1, wsem0, wsem1):
    w = _wid()
    cnt = SUP_BASE + jnp.where(w < SUP_EXTRA, 1, 0)
    ssems = (ssem0, ssem1)
    wsems = (wsem0, wsem1)

    def sb_of(i):
        return w + i * NW

    def stage(sb, buf):
        c0 = pl.multiple_of(sb * SBW, 128)
        for dt in range(8):
            pltpu.async_copy(
                wt.at[pl.ds(dt * 8, 8), pl.ds(c0, SBW)],
                sbuf.at[buf, pl.ds(dt * 8, 8)], ssems[buf])

    def wait_stage(buf):
        for dt in range(8):
            pltpu.make_async_copy(
                wt.at[pl.ds(0, 8), pl.ds(0, SBW)],
                sbuf.at[buf, pl.ds(dt * 8, 8)], ssems[buf]).wait()

    def transpose(buf, nvoc):
        for k in range(4):
            rows = jnp.arange(16, dtype=jnp.int32) + 16 * k

            def body(v, carry):
                cols = jnp.full((16,), v, dtype=jnp.int32)
                g = plsc.load_gather(sbuf.at[buf], [rows, cols])
                flat = v * 64 + 16 * k
                tbuf[buf, flat // 128, pl.ds(lax.rem(flat, 128), 16)] = g
                return carry

            lax.fori_loop(0, nvoc, body, 0)

    def step(i, buf):
        @pl.when(i < cnt)
        def _():
            wait_stage(buf)
            @pl.when(i >= 2)
            def _():
                pltpu.make_async_copy(
                    tbuf.at[buf], wrm.at[pl.ds(0, 64 * KSUP)],
                    wsems[buf]).wait()
            transpose(buf, SBW)
            @pl.when(i + 2 < cnt)
            def _():
                stage(sb_of(i + 2), buf)
            pltpu.async_copy(
                tbuf.at[buf],
                wrm.at[pl.ds(pl.multiple_of(sb_of(i) * (64 * KSUP), 64),
                             64 * KSUP)],
                wsems[buf])

    stage(sb_of(0), 0)
    @pl.when(cnt > 1)
    def _():
        stage(sb_of(1), 1)

    def pair(ii, carry):
        step(2 * ii, 0)
        step(2 * ii + 1, 1)
        return carry

    lax.fori_loop(0, (SUP_BASE + 2) // 2, pair, 0)

    @pl.when(cnt >= 1)
    def _():
        pltpu.make_async_copy(
            tbuf.at[0], wrm.at[pl.ds(0, 64 * KSUP)], wsem0).wait()
    @pl.when(cnt >= 2)
    def _():
        pltpu.make_async_copy(
            tbuf.at[1], wrm.at[pl.ds(0, 64 * KSUP)], wsem1).wait()

    @pl.when(w == NW - 1)
    def _():
        pltpu.sync_copy(wt_tail, sbuf.at[0])
        for k in range(4):
            rows = jnp.arange(16, dtype=jnp.int32) + 16 * k

            def tbody(v, carry):
                cols = jnp.full((16,), v, dtype=jnp.int32)
                g = plsc.load_gather(sbuf.at[0], [rows, cols])
                flat = v * 64 + 16 * k
                tbuf[0, flat // 128, pl.ds(lax.rem(flat, 128), 16)] = g
                return carry

            lax.fori_loop(0, 64, tbody, 0)
        pltpu.sync_copy(tbuf.at[0, pl.ds(0, 32)],
                        wrm.at[pl.ds(VBLK * 64, 32)])


def _kernel_b(xt, wrm, out5, ibuf, sidx, gbuf, tbuf,
              isem, gsem0, gsem1, osem0, osem1):
    w = _wid()
    gsems = (gsem0, gsem1)
    osems = (osem0, osem1)

    for lt in range(L_TILES):
        pltpu.async_copy(
            xt.at[pl.ds(lt * 8, 8), pl.ds(pl.multiple_of(w * 128, 128), 128)],
            ibuf.at[pl.ds(lt * 8, 8)], isem)
    for lt in range(L_TILES):
        pltpu.make_async_copy(
            xt.at[pl.ds(0, 8), pl.ds(0, 128)],
            ibuf.at[pl.ds(lt * 8, 8)], isem).wait()

    def make_sidx(l, buf):
        for k in range(8):
            v = ibuf[l, pl.ds(16 * k, 16)]
            sidx[buf, pl.ds(16 * k, 16)] = lax.shift_right_logical(v, 1)

    def issue_gather(buf):
        pltpu.async_copy(wrm.at[sidx.at[buf]], gbuf.at[buf], gsems[buf])

    def wait_gather(buf):
        pltpu.make_async_copy(wrm.at[sidx.at[buf]], gbuf.at[buf],
                              gsems[buf]).wait()

    def transpose_b(l, buf):
        for k in range(8):
            rows = jnp.arange(16, dtype=jnp.int32) + 16 * k
            v = ibuf[l, pl.ds(16 * k, 16)]
            pb = lax.shift_left(jnp.bitwise_and(v, 1), 6)
            for d in range(64):
                g = plsc.load_gather(gbuf.at[buf], [rows, pb + d])
                tbuf[buf, d // 8, d % 8, pl.ds(16 * k, 16)] = g

    def write_out(l, buf):
        for dt in range(8):
            pltpu.async_copy(tbuf.at[buf, dt], out5.at[l, dt, w], osems[buf])

    def wait_write(buf):
        for dt in range(8):
            pltpu.make_async_copy(tbuf.at[buf, dt],
                                  out5.at[0, dt, 0], osems[buf]).wait()

    make_sidx(0, 0)
    issue_gather(0)
    make_sidx(1, 1)
    issue_gather(1)

    def step(l, buf):
        wait_gather(buf)
        @pl.when(l >= 2)
        def _():
            wait_write(buf)
        transpose_b(l, buf)
        @pl.when(l + 2 < L)
        def _():
            make_sidx(l + 2, buf)
            issue_gather(buf)
        write_out(l, buf)

    def pair(ll, carry):
        step(2 * ll, 0)
        step(2 * ll + 1, 1)
        return carry

    lax.fori_loop(0, L // 2, pair, 0)
    wait_write(0)
    wait_write(1)


def kernel(x, W):
    mesh = plsc.VectorSubcoreMesh(core_axis_name="c", subcore_axis_name="s")
    wt = W.T
    wt_tail = jnp.concatenate(
        [wt[:, VBLK * 128:], jnp.zeros((64, SBW - 64), jnp.float32)], axis=1)
    xt = x.T.astype(jnp.int32)

    run_a = pl.kernel(
        _kernel_a,
        out_type=jax.ShapeDtypeStruct((V // 2, 128), jnp.float32),
        mesh=mesh,
        scratch_types=[
            pltpu.VMEM((2, 64, SBW), jnp.float32),
            pltpu.VMEM((2, 64 * KSUP, 128), jnp.float32),
            pltpu.SemaphoreType.DMA,
            pltpu.SemaphoreType.DMA,
            pltpu.SemaphoreType.DMA,
            pltpu.SemaphoreType.DMA,
        ],
        compiler_params=pltpu.CompilerParams(use_tc_tiling_on_sc=True, needs_layout_passes=False),
    )
    wrm = run_a(wt, wt_tail)

    run_b = pl.kernel(
        _kernel_b,
        out_type=jax.ShapeDtypeStruct((L, 8, BT, 8, 128), jnp.float32),
        mesh=mesh,
        scratch_types=[
            pltpu.VMEM((L, 128), jnp.int32),
            pltpu.VMEM((2, 128), jnp.int32),
            pltpu.VMEM((2, 128, 128), jnp.float32),
            pltpu.VMEM((2, 8, 8, 128), jnp.float32),
            pltpu.SemaphoreType.DMA,
            pltpu.SemaphoreType.DMA,
            pltpu.SemaphoreType.DMA,
            pltpu.SemaphoreType.DMA,
            pltpu.SemaphoreType.DMA,
        ],
        compiler_params=pltpu.CompilerParams(use_tc_tiling_on_sc=True, needs_layout_passes=False),
    )
    out5 = run_b(xt, wrm)
    return out5.transpose((2, 4, 0, 1, 3)).reshape(B, L, D)
